# baseline (speedup 1.0000x reference)
import functools

import jax
import jax.numpy as jnp
from jax import lax
from jax.experimental import pallas as pl
from jax.experimental.pallas import tpu as pltpu

_NOISE_LEVEL = 1e-05
_LANES = 1024


def _fused_kernel(state_ref, w_ref, noise_ref, out_ref, q_ref):
    k = pl.program_id(1)
    nk = pl.num_programs(1)
    d = jnp.dot(state_ref[...], w_ref[...], preferred_element_type=jnp.float32)

    @pl.when(k == 0)
    def _init():
        q_ref[...] = d

    @pl.when(k != 0)
    def _acc():
        q_ref[...] = q_ref[...] + d

    @pl.when(k == nk - 1)
    def _epilogue():
        tb, a = q_ref.shape
        nfr = tb * a // _LANES
        q = q_ref[...] + noise_ref[...]
        j = jnp.argmax(q, axis=-1).astype(jnp.int32)
        r = lax.iota(jnp.int32, tb)
        p = r * a + j
        fr = lax.shift_right_logical(p, 10)
        lane = lax.bitwise_and(p, _LANES - 1)
        fc_iota = lax.broadcasted_iota(jnp.int32, (nfr, tb), 0)
        m1t = (fc_iota == fr[None, :]).astype(jnp.bfloat16)
        lane_iota = lax.broadcasted_iota(jnp.int32, (tb, _LANES), 1)
        lmat = (lane_iota == lane[:, None]).astype(jnp.bfloat16)
        out_ref[...] = jnp.dot(m1t, lmat, preferred_element_type=jnp.float32)


@functools.partial(jax.jit, static_argnames=("tb", "tk"))
def _run(state, W, noise, tb, tk):
    B, d = state.shape
    A = W.shape[1]
    nfr = tb * A // _LANES
    grid = (B // tb, d // tk)
    flat = pl.pallas_call(
        _fused_kernel,
        grid=grid,
        in_specs=[
            pl.BlockSpec((tb, tk), lambda b, k: (b, k)),
            pl.BlockSpec((tk, A), lambda b, k: (k, 0)),
            pl.BlockSpec((tb, A), lambda b, k: (b, 0)),
        ],
        out_specs=pl.BlockSpec((nfr, _LANES), lambda b, k: (b, 0)),
        out_shape=jax.ShapeDtypeStruct((B * A // _LANES, _LANES), jnp.float32),
        scratch_shapes=[pltpu.VMEM((tb, A), jnp.float32)],
    )(state, W, noise)
    return jnp.reshape(flat, (B, A))


def kernel(state, W):
    B, d = state.shape
    A = W.shape[1]
    with jax.ensure_compile_time_eval():
        rand = jax.random.uniform(jax.random.key(1), (B, A), dtype=jnp.float32)
        noise = (rand * 2 - 1) * _NOISE_LEVEL
    return _run(state, W, noise, 1024, 1024)


# speedup vs baseline: 1.0266x; 1.0266x over previous
import functools

import jax
import jax.numpy as jnp
from jax import lax
from jax.experimental import pallas as pl
from jax.experimental.pallas import tpu as pltpu

_NOISE_LEVEL = 1e-05
_LANES = 1024


def _fused_kernel(state_ref, w_ref, noise_ref, out_ref, q_ref):
    k = pl.program_id(1)
    nk = pl.num_programs(1)
    d = jnp.dot(state_ref[...], w_ref[...], preferred_element_type=jnp.float32)

    @pl.when(k == 0)
    def _init():
        q_ref[...] = d

    @pl.when(k != 0)
    def _acc():
        q_ref[...] = q_ref[...] + d

    @pl.when(k == nk - 1)
    def _epilogue():
        tb, a = q_ref.shape
        nfr = tb * a // _LANES
        q = q_ref[...] + noise_ref[...]
        j = jnp.argmax(q, axis=-1).astype(jnp.int32)
        r = lax.iota(jnp.int32, tb)
        p = r * a + j
        fr = lax.shift_right_logical(p, 10)
        lane = lax.bitwise_and(p, _LANES - 1)
        fc_iota = lax.broadcasted_iota(jnp.int32, (nfr, tb), 0)
        m1t = (fc_iota == fr[None, :]).astype(jnp.bfloat16)
        lane_iota = lax.broadcasted_iota(jnp.int32, (tb, _LANES), 1)
        lmat = (lane_iota == lane[:, None]).astype(jnp.bfloat16)
        out_ref[...] = jnp.dot(m1t, lmat, preferred_element_type=jnp.float32)


@functools.partial(jax.jit, static_argnames=("tb", "tk"))
def _run(state, W, noise, tb, tk):
    B, d = state.shape
    A = W.shape[1]
    nfr = tb * A // _LANES
    grid = (B // tb, d // tk)
    flat = pl.pallas_call(
        _fused_kernel,
        grid=grid,
        in_specs=[
            pl.BlockSpec((tb, tk), lambda b, k: (b, k)),
            pl.BlockSpec((tk, A), lambda b, k: (k, 0)),
            pl.BlockSpec((tb, A), lambda b, k: (b, 0)),
        ],
        out_specs=pl.BlockSpec((nfr, _LANES), lambda b, k: (b, 0)),
        out_shape=jax.ShapeDtypeStruct((B * A // _LANES, _LANES), jnp.float32),
        scratch_shapes=[pltpu.VMEM((tb, A), jnp.float32)],
        compiler_params=pltpu.CompilerParams(vmem_limit_bytes=63 * 1024 * 1024),
    )(state, W, noise)
    return jnp.reshape(flat, (B, A))


def kernel(state, W):
    B, d = state.shape
    A = W.shape[1]
    with jax.ensure_compile_time_eval():
        rand = jax.random.uniform(jax.random.key(1), (B, A), dtype=jnp.float32)
        noise = (rand * 2 - 1) * _NOISE_LEVEL
    return _run(state, W, noise, 1024, 2048)


# speedup vs baseline: 1.0870x; 1.0589x over previous
import functools

import jax
import jax.numpy as jnp
from jax import lax
from jax.experimental import pallas as pl

_NOISE_LEVEL = 1e-05
_LANES = 1024


def _argmax_kernel(state_ref, w_ref, noise_ref, j_ref):
    q = jnp.dot(state_ref[...], w_ref[...], preferred_element_type=jnp.float32)
    q = q + noise_ref[...]
    j = jnp.argmax(q, axis=-1).astype(jnp.int32)
    j_ref[...] = jnp.broadcast_to(j[:, None], j_ref.shape)


def _scatter_kernel(j_ref, out_ref):
    nfr, lanes = out_ref.shape
    tb = j_ref.shape[0]
    a = nfr * lanes // tb
    j = j_ref[:, 0]
    r = lax.iota(jnp.int32, tb)
    p = r * a + j
    fr = lax.shift_right_logical(p, 10)
    lane = lax.bitwise_and(p, lanes - 1)
    fc_iota = lax.broadcasted_iota(jnp.int32, (nfr, tb), 0)
    m1t = (fc_iota == fr[None, :]).astype(jnp.bfloat16)
    lane_iota = lax.broadcasted_iota(jnp.int32, (tb, lanes), 1)
    lmat = (lane_iota == lane[:, None]).astype(jnp.bfloat16)
    out_ref[...] = jnp.dot(m1t, lmat, preferred_element_type=jnp.float32)


@functools.partial(jax.jit, static_argnames=("tb", "ts"))
def _run(state, W, noise, tb, ts):
    B, d = state.shape
    A = W.shape[1]
    j = pl.pallas_call(
        _argmax_kernel,
        grid=(B // tb,),
        in_specs=[
            pl.BlockSpec((tb, d), lambda i: (i, 0)),
            pl.BlockSpec((d, A), lambda i: (0, 0)),
            pl.BlockSpec((tb, A), lambda i: (i, 0)),
        ],
        out_specs=pl.BlockSpec((tb, 8), lambda i: (i, 0)),
        out_shape=jax.ShapeDtypeStruct((B, 8), jnp.int32),
    )(state, W, noise)
    nfr = ts * A // _LANES
    flat = pl.pallas_call(
        _scatter_kernel,
        grid=(B // ts,),
        in_specs=[pl.BlockSpec((ts, 8), lambda i: (i, 0))],
        out_specs=pl.BlockSpec((nfr, _LANES), lambda i: (i, 0)),
        out_shape=jax.ShapeDtypeStruct((B * A // _LANES, _LANES), jnp.float32),
    )(j)
    return jnp.reshape(flat, (B, A))


def kernel(state, W):
    B, d = state.shape
    A = W.shape[1]
    with jax.ensure_compile_time_eval():
        rand = jax.random.uniform(jax.random.key(1), (B, A), dtype=jnp.float32)
        noise = (rand * 2 - 1) * _NOISE_LEVEL
    return _run(state, W, noise, 512, 1024)


# speedup vs baseline: 1.6704x; 1.5367x over previous
import functools

import jax
import jax.numpy as jnp
from jax import lax
from jax.experimental import pallas as pl

_NOISE_LEVEL = 1e-05


def _fused_kernel(state_ref, w_ref, noise_ref, out_ref):
    q = jnp.dot(state_ref[...], w_ref[...], preferred_element_type=jnp.float32)
    q = q + noise_ref[...]
    j = jnp.argmax(q, axis=-1)
    iota = lax.broadcasted_iota(jnp.int32, q.shape, 1)
    out_ref[...] = (iota == j[:, None]).astype(jnp.float32)


@functools.partial(jax.jit, static_argnames=("tb",))
def _run(state, W, noise, tb):
    B, d = state.shape
    A = W.shape[1]
    grid = (B // tb,)
    return pl.pallas_call(
        _fused_kernel,
        grid=grid,
        in_specs=[
            pl.BlockSpec((tb, d), lambda i: (i, 0)),
            pl.BlockSpec((d, A), lambda i: (0, 0)),
            pl.BlockSpec((tb, A), lambda i: (i, 0)),
        ],
        out_specs=pl.BlockSpec((tb, A), lambda i: (i, 0)),
        out_shape=jax.ShapeDtypeStruct((B, A), jnp.float32),
    )(state, W, noise)


def kernel(state, W):
    B, d = state.shape
    A = W.shape[1]
    with jax.ensure_compile_time_eval():
        rand = jax.random.uniform(jax.random.key(1), (B, A), dtype=jnp.float32)
        noise = (rand * 2 - 1) * _NOISE_LEVEL
    tb = 512 if B % 512 == 0 else B
    return _run(state, W, noise, tb)
